# baseline (speedup 1.0000x reference)
import functools

import jax
import jax.numpy as jnp
from jax import lax
from jax.experimental import pallas as pl
from jax.experimental.pallas import tpu as pltpu
from jax.experimental.pallas import tpu_sc as plsc

_INFO = plsc.get_sparse_core_info()
_NC = _INFO.num_cores
_NS = _INFO.num_subcores
_NW = _NC * _NS
_K = 8


@functools.lru_cache(maxsize=None)
def _make_gather(B, S, V, D):
    N = B * S
    assert N % _NW == 0
    b_per_w = N // _NW
    assert S % b_per_w == 0
    w_per_row = S // b_per_w
    assert b_per_w % _K == 0
    n_groups = b_per_w // _K

    mesh = plsc.VectorSubcoreMesh(core_axis_name="c", subcore_axis_name="s")

    @functools.partial(
        pl.kernel,
        mesh=mesh,
        out_type=jax.ShapeDtypeStruct((B, S, D), jnp.float32),
        scratch_types=(
            [
                pltpu.VMEM((b_per_w,), jnp.int32),
                pltpu.VMEM_SHARED((_NS, b_per_w), jnp.int32),
                pltpu.SMEM((b_per_w,), jnp.int32),
            ]
            + [pltpu.SemaphoreType.DMA for _ in range(_K)]
        ),
    )
    def gather(table_hbm, idx_hbm, out_hbm, idx_v, idx_sp, idx_s, *sems):
        cid = lax.axis_index("c")
        sid = lax.axis_index("s")
        wid = sid * _NC + cid
        row = wid // w_per_row
        col = (wid % w_per_row) * b_per_w
        pltpu.sync_copy(idx_hbm.at[row, pl.ds(col, b_per_w)], idx_v)
        pltpu.sync_copy(idx_v, idx_sp.at[sid])
        pltpu.sync_copy(idx_sp.at[sid], idx_s)

        def issue(i, b):
            pltpu.async_copy(table_hbm.at[idx_s[i]],
                             out_hbm.at[row, col + i], sems[b])

        def drain(i, b):
            pltpu.make_async_copy(table_hbm.at[0],
                                  out_hbm.at[row, col + i], sems[b]).wait()

        for b in range(_K):
            issue(b, b)

        def body(g, carry):
            for b in range(_K):
                i = g * _K + b
                drain(i - _K, b)
                issue(i, b)
            return carry

        lax.fori_loop(1, n_groups, body, 0)
        for b in range(_K):
            drain(b_per_w - _K + b, b)

    return gather


def kernel(sequence, token_table):
    B, S = sequence.shape
    V, D = token_table.shape
    idx = sequence.astype(jnp.int32)
    return _make_gather(B, S, V, D)(token_table, idx)


# speedup vs baseline: 20.8678x; 20.8678x over previous
import functools

import jax
import jax.numpy as jnp
from jax import lax
from jax.experimental import pallas as pl
from jax.experimental.pallas import tpu as pltpu
from jax.experimental.pallas import tpu_sc as plsc

_INFO = plsc.get_sparse_core_info()
_NC = _INFO.num_cores
_NS = _INFO.num_subcores
_NW = _NC * _NS


@functools.lru_cache(maxsize=None)
def _make_gather(B, S, V, D):
    N = B * S
    assert N % _NW == 0
    b_per_w = N // _NW
    assert S % b_per_w == 0
    w_per_row = S // b_per_w
    chunk = min(b_per_w, 16)
    assert b_per_w % chunk == 0
    n_chunks = b_per_w // chunk
    nbuf = min(n_chunks, 8)

    mesh = plsc.VectorSubcoreMesh(core_axis_name="c", subcore_axis_name="s")

    @functools.partial(
        pl.kernel,
        mesh=mesh,
        out_type=jax.ShapeDtypeStruct((B, S, D), jnp.float32),
        scratch_types=(
            [pltpu.VMEM((b_per_w,), jnp.int32)]
            + [pltpu.VMEM((chunk, D), jnp.float32) for _ in range(nbuf)]
            + [pltpu.SemaphoreType.DMA for _ in range(2 * nbuf)]
        ),
    )
    def gather(table_hbm, idx_hbm, out_hbm, idx_v, *bufs_and_sems):
        rows = bufs_and_sems[:nbuf]
        isems = bufs_and_sems[nbuf:2 * nbuf]
        osems = bufs_and_sems[2 * nbuf:]
        wid = lax.axis_index("s") * _NC + lax.axis_index("c")
        row = wid // w_per_row
        col = (wid % w_per_row) * b_per_w
        pltpu.sync_copy(idx_hbm.at[row, pl.ds(col, b_per_w)], idx_v)

        def start_gather(g):
            b = g % nbuf
            return pltpu.async_copy(
                table_hbm.at[idx_v.at[pl.ds(g * chunk, chunk)]],
                rows[b], isems[b])

        in_cp = [None] * n_chunks
        out_cp = [None] * n_chunks
        for g in range(min(nbuf - 1, n_chunks)):
            in_cp[g] = start_gather(g)
        for g in range(n_chunks):
            b = g % nbuf
            ng = g + nbuf - 1
            if ng < n_chunks:
                if g >= 1:
                    out_cp[g - 1].wait()
                in_cp[ng] = start_gather(ng)
            in_cp[g].wait()
            out_cp[g] = pltpu.async_copy(
                rows[b], out_hbm.at[row, pl.ds(col + g * chunk, chunk)],
                osems[b])
        for g in range(max(0, n_chunks - nbuf + 1), n_chunks):
            out_cp[g].wait()

    return gather


def kernel(sequence, token_table):
    B, S = sequence.shape
    V, D = token_table.shape
    idx = sequence.astype(jnp.int32)
    return _make_gather(B, S, V, D)(token_table, idx)


# speedup vs baseline: 21.0761x; 1.0100x over previous
import functools

import jax
import jax.numpy as jnp
from jax import lax
from jax.experimental import pallas as pl
from jax.experimental.pallas import tpu as pltpu
from jax.experimental.pallas import tpu_sc as plsc

_INFO = plsc.get_sparse_core_info()
_NC = _INFO.num_cores
_NS = _INFO.num_subcores
_NW = _NC * _NS


@functools.lru_cache(maxsize=None)
def _make_gather(B, S, V, D):
    N = B * S
    assert N % _NW == 0
    b_per_w = N // _NW
    assert S % b_per_w == 0
    w_per_row = S // b_per_w
    chunk = min(b_per_w, 32)
    assert b_per_w % chunk == 0
    n_chunks = b_per_w // chunk
    nbuf = min(n_chunks, 5)

    mesh = plsc.VectorSubcoreMesh(core_axis_name="c", subcore_axis_name="s")

    @functools.partial(
        pl.kernel,
        mesh=mesh,
        out_type=jax.ShapeDtypeStruct((B, S, D), jnp.float32),
        scratch_types=(
            [pltpu.VMEM((b_per_w,), jnp.int32)]
            + [pltpu.VMEM((chunk, D), jnp.float32) for _ in range(nbuf)]
            + [pltpu.SemaphoreType.DMA for _ in range(2 * nbuf)]
        ),
    )
    def gather(table_hbm, idx_hbm, out_hbm, idx_v, *bufs_and_sems):
        rows = bufs_and_sems[:nbuf]
        isems = bufs_and_sems[nbuf:2 * nbuf]
        osems = bufs_and_sems[2 * nbuf:]
        wid = lax.axis_index("s") * _NC + lax.axis_index("c")
        row = wid // w_per_row
        col = (wid % w_per_row) * b_per_w
        pltpu.sync_copy(idx_hbm.at[row, pl.ds(col, b_per_w)], idx_v)

        def start_gather(g):
            b = g % nbuf
            return pltpu.async_copy(
                table_hbm.at[idx_v.at[pl.ds(g * chunk, chunk)]],
                rows[b], isems[b])

        in_cp = [None] * n_chunks
        out_cp = [None] * n_chunks
        for g in range(min(nbuf - 1, n_chunks)):
            in_cp[g] = start_gather(g)
        for g in range(n_chunks):
            b = g % nbuf
            ng = g + nbuf - 1
            if ng < n_chunks:
                if g >= 1:
                    out_cp[g - 1].wait()
                in_cp[ng] = start_gather(ng)
            in_cp[g].wait()
            out_cp[g] = pltpu.async_copy(
                rows[b], out_hbm.at[row, pl.ds(col + g * chunk, chunk)],
                osems[b])
        for g in range(max(0, n_chunks - nbuf + 1), n_chunks):
            out_cp[g].wait()

    return gather


def kernel(sequence, token_table):
    B, S = sequence.shape
    V, D = token_table.shape
    idx = sequence.astype(jnp.int32)
    return _make_gather(B, S, V, D)(token_table, idx)
